# baseline (speedup 1.0000x reference)
import functools

import jax
import jax.numpy as jnp
from jax import lax
from jax.experimental import pallas as pl
from jax.experimental.pallas import tpu as pltpu
from jax.experimental.pallas import tpu_sc as plsc

B = 16384
F = 26
V = 100000
E = 64
NNF = 13

NPAIR = (7, 6)
PBASE = (0, 7)
VB = 12544
NVB = -(-V // VB)

_NC, _NS = 2, 16
_NW = _NC * _NS
_CHUNK = 128
_NBUF = 4


_DN_T = (((0,), (0,)), ((), ()))


def _tr_body(a_ref, b_ref, out_ref):
    ident = (
        lax.broadcasted_iota(jnp.int32, (E, E), 0)
        == lax.broadcasted_iota(jnp.int32, (E, E), 1)
    ).astype(jnp.float32)
    a_t = lax.dot_general(a_ref[0, :, :], ident, _DN_T,
                          preferred_element_type=jnp.float32)
    b_t = lax.dot_general(b_ref[0, :, :], ident, _DN_T,
                          preferred_element_type=jnp.float32)
    out_ref[0, :, :] = jnp.concatenate([a_t, b_t], axis=1)


def _make_transpose(group):
    npair = NPAIR[group]
    base = PBASE[group]
    return pl.pallas_call(
        _tr_body,
        grid=(npair, NVB),
        in_specs=[
            pl.BlockSpec((1, E, VB), lambda g, j: (2 * (base + g), 0, j)),
            pl.BlockSpec((1, E, VB), lambda g, j: (2 * (base + g) + 1, 0, j)),
        ],
        out_specs=pl.BlockSpec((1, VB, 2 * E), lambda g, j: (g, j, 0)),
        out_shape=jax.ShapeDtypeStruct((npair, V, 2 * E), jnp.float32),
        compiler_params=pltpu.CompilerParams(
            dimension_semantics=("arbitrary", "arbitrary"),
        ),
    )


_transpose_calls = [_make_transpose(g) for g in range(2)]

_sc_mesh = plsc.VectorSubcoreMesh(
    core_axis_name="c", subcore_axis_name="s", num_cores=_NC, num_subcores=_NS
)


def _make_sc_gather(rows):
    nchunk = rows // (_NW * _CHUNK)
    rows_per_w = rows // _NW

    @functools.partial(
        pl.kernel,
        mesh=_sc_mesh,
        out_type=jax.ShapeDtypeStruct((rows, E), jnp.float32),
        scratch_types=[
            pltpu.VMEM((nchunk, _CHUNK), jnp.int32),
            pltpu.VMEM((_CHUNK, E), jnp.float32),
            pltpu.VMEM((_CHUNK, E), jnp.float32),
            pltpu.VMEM((_CHUNK, E), jnp.float32),
            pltpu.VMEM((_CHUNK, E), jnp.float32),
            pltpu.SemaphoreType.DMA,
            pltpu.SemaphoreType.DMA,
            pltpu.SemaphoreType.DMA,
            pltpu.SemaphoreType.DMA,
        ],
        compiler_params=pltpu.CompilerParams(use_tc_tiling_on_sc=False),
    )
    def _sc_gather(table_hbm, idx_hbm, out_hbm,
                   idx_v, b0, b1, b2, b3, s0, s1, s2, s3):
        wid = lax.axis_index("s") * _NC + lax.axis_index("c")
        chunk_base = wid * nchunk
        row_base = wid * rows_per_w
        bufs = (b0, b1, b2, b3)
        sems = (s0, s1, s2, s3)

        pltpu.sync_copy(idx_hbm.at[pl.ds(chunk_base, nchunk)], idx_v)

        for b in range(_NBUF):
            pltpu.make_async_copy(
                table_hbm.at[idx_v.at[b]], bufs[b], sems[b]
            ).start()

        def body(g, carry):
            for b in range(_NBUF):
                j = g * _NBUF + b
                pltpu.make_async_copy(
                    table_hbm.at[idx_v.at[j]], bufs[b], sems[b]
                ).wait()
                pltpu.sync_copy(
                    bufs[b], out_hbm.at[pl.ds(row_base + j * _CHUNK, _CHUNK)]
                )
                nj = j + _NBUF

                @pl.when(nj < nchunk)
                def _():
                    pltpu.make_async_copy(
                        table_hbm.at[idx_v.at[nj]], bufs[b], sems[b]
                    ).start()

            return carry

        lax.fori_loop(0, nchunk // _NBUF, body, 0)

    return _sc_gather


_sc_gathers = [_make_sc_gather(B * 2 * NPAIR[g]) for g in range(2)]

_BB = 1024
_NB = B // _BB
_NPALL = NPAIR[0] + NPAIR[1]


def _mma_body(*refs):
    g_refs = refs[:NPAIR[0]]
    xn_ref, w_ref, wnum_ref, bnum_ref, wtail_ref, bfin_ref, out_ref = \
        refs[NPAIR[0]:]
    num = (
        jnp.dot(xn_ref[...], wnum_ref[...], preferred_element_type=jnp.float32)
        + bnum_ref[...]
    )
    acc = jnp.dot(num, wtail_ref[...], preferred_element_type=jnp.float32)
    acc = acc + bfin_ref[...]
    for t in range(NPAIR[0]):
        acc = acc + jnp.dot(g_refs[t][...], w_ref[t, :, :],
                            preferred_element_type=jnp.float32)
    out_ref[...] = acc


def _mmb_body(*refs):
    g_refs = refs[:NPAIR[1]]
    prev_ref, w_ref, out_ref = refs[NPAIR[1]:]
    acc = prev_ref[...]
    for t in range(NPAIR[1]):
        acc = acc + jnp.dot(g_refs[t][...], w_ref[t, :, :],
                            preferred_element_type=jnp.float32)
    out_ref[...] = acc


def _g_spec(t):
    return pl.BlockSpec((_BB, 2 * E), lambda i, t=t: (t * _NB + i, 0))


_tc_matmul_a = pl.pallas_call(
    _mma_body,
    grid=(_NB,),
    in_specs=(
        [_g_spec(t) for t in range(NPAIR[0])]
        + [
            pl.BlockSpec((_BB, NNF), lambda i: (i, 0)),
            pl.BlockSpec((NPAIR[0], 2 * E, E), lambda i: (0, 0, 0)),
            pl.BlockSpec((NNF, E), lambda i: (0, 0)),
            pl.BlockSpec((1, E), lambda i: (0, 0)),
            pl.BlockSpec((E, E), lambda i: (0, 0)),
            pl.BlockSpec((1, E), lambda i: (0, 0)),
        ]
    ),
    out_specs=pl.BlockSpec((_BB, E), lambda i: (i, 0)),
    out_shape=jax.ShapeDtypeStruct((B, E), jnp.float32),
    compiler_params=pltpu.CompilerParams(
        dimension_semantics=("arbitrary",),
    ),
)

_tc_matmul_b = pl.pallas_call(
    _mmb_body,
    grid=(_NB,),
    in_specs=(
        [_g_spec(t) for t in range(NPAIR[1])]
        + [
            pl.BlockSpec((_BB, E), lambda i: (i, 0)),
            pl.BlockSpec((NPAIR[1], 2 * E, E), lambda i: (0, 0, 0)),
        ]
    ),
    out_specs=pl.BlockSpec((_BB, E), lambda i: (i, 0)),
    out_shape=jax.ShapeDtypeStruct((B, E), jnp.float32),
    compiler_params=pltpu.CompilerParams(
        dimension_semantics=("arbitrary",),
    ),
)


def kernel(X_cat, X_num, tables, W_num, b_num, W_final, b_final):
    tt = jnp.transpose(tables, (0, 2, 1))
    xc = X_cat.astype(jnp.int32)
    g2 = []
    col_base = 0
    for g in range(2):
        npair = NPAIR[g]
        nf = 2 * npair
        packed = _transpose_calls[g](tt, tt)
        flat = packed.reshape(npair * V * 2, E)
        xg3 = xc[:, col_base:col_base + nf].reshape(B, npair, 2)
        xg3 = jnp.transpose(xg3, (1, 0, 2))
        idx = (
            2 * xg3
            + (2 * V * jnp.arange(npair, dtype=jnp.int32))[:, None, None]
            + jnp.arange(2, dtype=jnp.int32)[None, None, :]
        )
        idx = idx.reshape(B * nf // _CHUNK, _CHUNK)
        gath = _sc_gathers[g](flat, idx)
        g2.append(gath.reshape(npair * B, 2 * E))
        col_base += nf
    w2a = W_final[: 2 * NPAIR[0] * E].reshape(NPAIR[0], 2 * E, E)
    w2b = W_final[2 * NPAIR[0] * E: 2 * _NPALL * E].reshape(NPAIR[1], 2 * E, E)
    out0 = _tc_matmul_a(
        *([g2[0]] * NPAIR[0]),
        X_num,
        w2a,
        W_num,
        b_num.reshape(1, E),
        W_final[2 * _NPALL * E:],
        b_final.reshape(1, E),
    )
    return _tc_matmul_b(*([g2[1]] * NPAIR[1]), out0, w2b)


# speedup vs baseline: 1.0174x; 1.0174x over previous
import functools

import jax
import jax.numpy as jnp
from jax import lax
from jax.experimental import pallas as pl
from jax.experimental.pallas import tpu as pltpu
from jax.experimental.pallas import tpu_sc as plsc

B = 16384
F = 26
V = 100000
E = 64
NNF = 13

NPAIR = (7, 6)
PBASE = (0, 7)
VB = 12544
NVB = -(-V // VB)

_NC, _NS = 2, 16
_NW = _NC * _NS
_CHUNK = 128
_NBUF = 4


_DN_T = (((0,), (0,)), ((), ()))


def _tr_body(a_ref, b_ref, out_ref):
    ident = (
        lax.broadcasted_iota(jnp.int32, (E, E), 0)
        == lax.broadcasted_iota(jnp.int32, (E, E), 1)
    ).astype(jnp.float32)
    a_t = lax.dot_general(a_ref[0, :, :], ident, _DN_T,
                          preferred_element_type=jnp.float32)
    b_t = lax.dot_general(b_ref[0, :, :], ident, _DN_T,
                          preferred_element_type=jnp.float32)
    out_ref[0, :, :] = jnp.concatenate([a_t, b_t], axis=1)


def _make_transpose(group):
    npair = NPAIR[group]
    base = PBASE[group]
    return pl.pallas_call(
        _tr_body,
        grid=(npair, NVB),
        in_specs=[
            pl.BlockSpec((1, E, VB), lambda g, j: (2 * (base + g), 0, j)),
            pl.BlockSpec((1, E, VB), lambda g, j: (2 * (base + g) + 1, 0, j)),
        ],
        out_specs=pl.BlockSpec((1, VB, 2 * E), lambda g, j: (g, j, 0)),
        out_shape=jax.ShapeDtypeStruct((npair, V, 2 * E), jnp.float32),
        compiler_params=pltpu.CompilerParams(
            dimension_semantics=("arbitrary", "arbitrary"),
        ),
    )


_transpose_calls = [_make_transpose(g) for g in range(2)]

_sc_mesh = plsc.VectorSubcoreMesh(
    core_axis_name="c", subcore_axis_name="s", num_cores=_NC, num_subcores=_NS
)


def _make_sc_gather(rows):
    nchunk = rows // (_NW * _CHUNK)
    rows_per_w = rows // _NW

    @functools.partial(
        pl.kernel,
        mesh=_sc_mesh,
        out_type=jax.ShapeDtypeStruct((rows, E), jnp.float32),
        scratch_types=[
            pltpu.VMEM((nchunk, _CHUNK), jnp.int32),
            pltpu.VMEM((_CHUNK, E), jnp.float32),
            pltpu.VMEM((_CHUNK, E), jnp.float32),
            pltpu.VMEM((_CHUNK, E), jnp.float32),
            pltpu.VMEM((_CHUNK, E), jnp.float32),
            pltpu.SemaphoreType.DMA,
            pltpu.SemaphoreType.DMA,
            pltpu.SemaphoreType.DMA,
            pltpu.SemaphoreType.DMA,
        ],
        compiler_params=pltpu.CompilerParams(use_tc_tiling_on_sc=False),
    )
    def _sc_gather(table_hbm, idx_hbm, out_hbm,
                   idx_v, b0, b1, b2, b3, s0, s1, s2, s3):
        wid = lax.axis_index("s") * _NC + lax.axis_index("c")
        chunk_base = wid * nchunk
        row_base = wid * rows_per_w
        bufs = (b0, b1, b2, b3)
        sems = (s0, s1, s2, s3)

        pltpu.sync_copy(idx_hbm.at[pl.ds(chunk_base, nchunk)], idx_v)

        for b in range(_NBUF):
            pltpu.make_async_copy(
                table_hbm.at[idx_v.at[b]], bufs[b], sems[b]
            ).start()

        def body(g, carry):
            for b in range(_NBUF):
                j = g * _NBUF + b
                pltpu.make_async_copy(
                    table_hbm.at[idx_v.at[j]], bufs[b], sems[b]
                ).wait()
                pltpu.sync_copy(
                    bufs[b], out_hbm.at[pl.ds(row_base + j * _CHUNK, _CHUNK)]
                )
                nj = j + _NBUF

                @pl.when(nj < nchunk)
                def _():
                    pltpu.make_async_copy(
                        table_hbm.at[idx_v.at[nj]], bufs[b], sems[b]
                    ).start()

            return carry

        lax.fori_loop(0, nchunk // _NBUF, body, 0)

    return _sc_gather


_sc_gathers = [_make_sc_gather(B * 2 * NPAIR[g]) for g in range(2)]

_BB = 1024
_NB = B // _BB
_NPALL = NPAIR[0] + NPAIR[1]


def _mm_body(*refs):
    g_refs = refs[:_NPALL]
    xn_ref, w_ref, wnum_ref, bnum_ref, wtail_ref, bfin_ref, out_ref = \
        refs[_NPALL:]
    num = (
        jnp.dot(xn_ref[...], wnum_ref[...], preferred_element_type=jnp.float32)
        + bnum_ref[...]
    )
    acc = jnp.dot(num, wtail_ref[...], preferred_element_type=jnp.float32)
    acc = acc + bfin_ref[...]
    for t in range(_NPALL):
        acc = acc + jnp.dot(g_refs[t][...], w_ref[t, :, :],
                            preferred_element_type=jnp.float32)
    out_ref[...] = acc


def _g_spec(t):
    return pl.BlockSpec((_BB, 2 * E), lambda i, t=t: (t * _NB + i, 0))


_tc_matmul = pl.pallas_call(
    _mm_body,
    grid=(_NB,),
    in_specs=(
        [_g_spec(t) for t in range(NPAIR[0])]
        + [_g_spec(t) for t in range(NPAIR[1])]
        + [
            pl.BlockSpec((_BB, NNF), lambda i: (i, 0)),
            pl.BlockSpec((_NPALL, 2 * E, E), lambda i: (0, 0, 0)),
            pl.BlockSpec((NNF, E), lambda i: (0, 0)),
            pl.BlockSpec((1, E), lambda i: (0, 0)),
            pl.BlockSpec((E, E), lambda i: (0, 0)),
            pl.BlockSpec((1, E), lambda i: (0, 0)),
        ]
    ),
    out_specs=pl.BlockSpec((_BB, E), lambda i: (i, 0)),
    out_shape=jax.ShapeDtypeStruct((B, E), jnp.float32),
    compiler_params=pltpu.CompilerParams(
        dimension_semantics=("arbitrary",),
    ),
)


def kernel(X_cat, X_num, tables, W_num, b_num, W_final, b_final):
    tt = jnp.transpose(tables, (0, 2, 1))
    xc = X_cat.astype(jnp.int32)
    g2 = []
    col_base = 0
    for g in range(2):
        npair = NPAIR[g]
        nf = 2 * npair
        packed = _transpose_calls[g](tt, tt)
        flat = packed.reshape(npair * V * 2, E)
        xg3 = xc[:, col_base:col_base + nf].reshape(B, npair, 2)
        xg3 = jnp.transpose(xg3, (1, 0, 2))
        idx = (
            2 * xg3
            + (2 * V * jnp.arange(npair, dtype=jnp.int32))[:, None, None]
            + jnp.arange(2, dtype=jnp.int32)[None, None, :]
        )
        idx = idx.reshape(B * nf // _CHUNK, _CHUNK)
        gath = _sc_gathers[g](flat, idx)
        g2.append(gath.reshape(npair * B, 2 * E))
        col_base += nf
    w2 = jnp.concatenate(
        [
            W_final[: 2 * NPAIR[0] * E].reshape(NPAIR[0], 2 * E, E),
            W_final[2 * NPAIR[0] * E: 2 * _NPALL * E].reshape(
                NPAIR[1], 2 * E, E
            ),
        ],
        axis=0,
    )
    return _tc_matmul(
        *([g2[0]] * NPAIR[0]),
        *([g2[1]] * NPAIR[1]),
        X_num,
        w2,
        W_num,
        b_num.reshape(1, E),
        W_final[2 * _NPALL * E:],
        b_final.reshape(1, E),
    )
